# baseline (speedup 1.0000x reference)
import jax
import jax.numpy as jnp
from jax.experimental import pallas as pl
from jax.experimental.pallas import tpu as pltpu


def _dma_body(perm_ref, x_ref, o_ref, sem):
    n = x_ref.shape[0]

    def issue(i, carry):
        pltpu.make_async_copy(x_ref.at[perm_ref[i]], o_ref.at[i], sem).start()
        return carry

    jax.lax.fori_loop(0, n, issue, 0)

    def drain(i, carry):
        pltpu.make_async_copy(x_ref.at[perm_ref[i]], o_ref.at[i], sem).wait()
        return carry

    jax.lax.fori_loop(0, n, drain, 0)


def kernel(x, perm):
    grid_spec = pltpu.PrefetchScalarGridSpec(
        num_scalar_prefetch=1,
        grid=(1,),
        in_specs=[pl.BlockSpec(memory_space=pl.ANY)],
        out_specs=pl.BlockSpec(memory_space=pl.ANY),
        scratch_shapes=[pltpu.SemaphoreType.DMA],
    )
    return pl.pallas_call(
        _dma_body,
        grid_spec=grid_spec,
        out_shape=jax.ShapeDtypeStruct(x.shape, x.dtype),
    )(perm.astype(jnp.int32), x)


# speedup vs baseline: 12.9619x; 12.9619x over previous
import jax
import jax.numpy as jnp
from jax.experimental import pallas as pl
from jax.experimental.pallas import tpu as pltpu


def _copy_body(perm_ref, x_ref, o_ref):
    del perm_ref
    o_ref[...] = x_ref[...]


def kernel(x, perm):
    n, h, w = x.shape
    bh = 128
    grid_spec = pltpu.PrefetchScalarGridSpec(
        num_scalar_prefetch=1,
        grid=(n, h // bh),
        in_specs=[
            pl.BlockSpec((1, bh, w), lambda i, j, perm_ref: (perm_ref[i], j, 0)),
        ],
        out_specs=pl.BlockSpec((1, bh, w), lambda i, j, perm_ref: (i, j, 0)),
    )
    return pl.pallas_call(
        _copy_body,
        grid_spec=grid_spec,
        out_shape=jax.ShapeDtypeStruct(x.shape, x.dtype),
    )(perm.astype(jnp.int32), x)


# speedup vs baseline: 47.4162x; 3.6581x over previous
import jax
import jax.numpy as jnp
from jax.experimental import pallas as pl
from jax.experimental.pallas import tpu as pltpu

_R = 4


def _copy_body(perm_ref, *refs):
    del perm_ref
    x_refs = refs[:_R]
    o_ref = refs[_R]
    for r in range(_R):
        o_ref[r] = x_refs[r][0]


def kernel(x, perm):
    n, h, w = x.shape
    in_specs = [
        pl.BlockSpec(
            (1, h, w),
            lambda i, perm_ref, r=r: (perm_ref[i * _R + r], 0, 0),
        )
        for r in range(_R)
    ]
    grid_spec = pltpu.PrefetchScalarGridSpec(
        num_scalar_prefetch=1,
        grid=(n // _R,),
        in_specs=in_specs,
        out_specs=pl.BlockSpec((_R, h, w), lambda i, perm_ref: (i, 0, 0)),
    )
    return pl.pallas_call(
        _copy_body,
        grid_spec=grid_spec,
        out_shape=jax.ShapeDtypeStruct(x.shape, x.dtype),
    )(perm.astype(jnp.int32), *([x] * _R))


# speedup vs baseline: 48.3946x; 1.0206x over previous
import jax
import jax.numpy as jnp
from jax.experimental import pallas as pl
from jax.experimental.pallas import tpu as pltpu

_R = 8


def _copy_body(perm_ref, *refs):
    del perm_ref
    x_refs = refs[:_R]
    o_ref = refs[_R]
    for r in range(_R):
        o_ref[r] = x_refs[r][0]


def kernel(x, perm):
    n, h, w = x.shape
    in_specs = [
        pl.BlockSpec(
            (1, h, w),
            lambda i, perm_ref, r=r: (perm_ref[i * _R + r], 0, 0),
        )
        for r in range(_R)
    ]
    grid_spec = pltpu.PrefetchScalarGridSpec(
        num_scalar_prefetch=1,
        grid=(n // _R,),
        in_specs=in_specs,
        out_specs=pl.BlockSpec((_R, h, w), lambda i, perm_ref: (i, 0, 0)),
    )
    return pl.pallas_call(
        _copy_body,
        grid_spec=grid_spec,
        out_shape=jax.ShapeDtypeStruct(x.shape, x.dtype),
    )(perm.astype(jnp.int32), *([x] * _R))
